# baseline (device time: 96542 ns/iter reference)
import jax
import jax.numpy as jnp
from jax import lax
from jax.experimental import pallas as pl
from jax.experimental.pallas import tpu as pltpu

N_RING = 8
S = 8

_CW_HOPS = {s: (4 if s < S // 2 else 3) for s in range(S)}
_CCW_HOPS = {s: (3 if s < S // 2 else 4) for s in range(S)}

_CW_IDX = {(h, s): i for i, (h, s) in enumerate(
    (h, s) for h in range(4) for s in range(S) if h < _CW_HOPS[s])}
_CCW_IDX = {(h, s): i for i, (h, s) in enumerate(
    (h, s) for h in range(4) for s in range(S) if h < _CCW_HOPS[s])}


def kernel(x):
    m, n = x.shape
    chunk = m // N_RING
    sub = chunk // S

    def body(x_hbm, out_hbm, gather, xchunk, peer_chunk, send_buf,
             local_sem, hbm_sem, y_send, y_recv,
             cw_send, cw_recv, ccw_send, ccw_recv):
        my_x = lax.axis_index("x")
        my_y = lax.axis_index("y")
        my_z = lax.axis_index("z")

        p = jnp.where(my_x == 0, my_z, 7 - my_z)

        def ring_xz(q):
            q = q % N_RING
            return jnp.where(q <= 3, 0, 1), jnp.where(q <= 3, q, 7 - q)

        nxt_x, nxt_z = ring_xz(p + 1)
        prv_x, prv_z = ring_xz(p - 1)

        y_peer = (my_x, 1 - my_y, my_z)
        nxt = (nxt_x, my_y, nxt_z)
        prv = (prv_x, my_y, prv_z)

        cp = pltpu.make_async_copy(
            x_hbm.at[pl.ds(p * chunk, chunk)], xchunk, local_sem)
        cp.start()

        bar = pltpu.get_barrier_semaphore()
        for dev in (y_peer, nxt, prv):
            pl.semaphore_signal(bar, inc=1, device_id=dev,
                                device_id_type=pl.DeviceIdType.MESH)
        pl.semaphore_wait(bar, 3)

        cp.wait()

        y_ex = []
        for s in range(S):
            srows = pl.ds(s * sub, sub)
            send_buf[srows, :] = xchunk[srows, :].astype(jnp.bfloat16)
            ex = pltpu.make_async_remote_copy(
                src_ref=send_buf.at[srows],
                dst_ref=peer_chunk.at[srows],
                send_sem=y_send.at[s], recv_sem=y_recv.at[s],
                device_id=y_peer, device_id_type=pl.DeviceIdType.MESH)
            ex.start()
            y_ex.append(ex)

        def mk(direction, h, s):
            if direction == "cw":
                c = (p - h) % N_RING
                dev, ssem, rsem, i = nxt, cw_send, cw_recv, _CW_IDX[(h, s)]
            else:
                c = (p + h) % N_RING
                dev, ssem, rsem, i = prv, ccw_send, ccw_recv, _CCW_IDX[(h, s)]
            off = c * chunk + s * sub
            return pltpu.make_async_remote_copy(
                src_ref=gather.at[pl.ds(off, sub)],
                dst_ref=gather.at[pl.ds(off, sub)],
                send_sem=ssem.at[i], recv_sem=rsem.at[i],
                device_id=dev, device_id_type=pl.DeviceIdType.MESH)

        hbm_copies = []

        def flush(c, s):
            off = c * chunk + s * sub
            d = pltpu.make_async_copy(
                gather.at[pl.ds(off, sub)], out_hbm.at[pl.ds(off, sub)],
                hbm_sem)
            d.start()
            hbm_copies.append(d)

        cws, ccws = {}, {}
        for s in range(S):
            y_ex[s].wait()
            rows = pl.ds(p * chunk + s * sub, sub)
            srows = pl.ds(s * sub, sub)
            gather[rows, :] = (
                xchunk[srows, :] + peer_chunk[srows, :].astype(jnp.float32)
            ).astype(jnp.bfloat16)
            cws[(0, s)] = mk("cw", 0, s)
            cws[(0, s)].start()
            ccws[(0, s)] = mk("ccw", 0, s)
            ccws[(0, s)].start()
            flush(p, s)

        for h in range(1, 4):
            for s in range(S):
                if h < _CW_HOPS[s]:
                    cws[(h - 1, s)].wait_recv()
                    cws[(h, s)] = mk("cw", h, s)
                    cws[(h, s)].start()
                    flush((p - h) % N_RING, s)
                if h < _CCW_HOPS[s]:
                    ccws[(h - 1, s)].wait_recv()
                    ccws[(h, s)] = mk("ccw", h, s)
                    ccws[(h, s)].start()
                    flush((p + h) % N_RING, s)

        for s in range(S):
            cws[(_CW_HOPS[s] - 1, s)].wait_recv()
            flush((p - _CW_HOPS[s]) % N_RING, s)
            ccws[(_CCW_HOPS[s] - 1, s)].wait_recv()
            flush((p + _CCW_HOPS[s]) % N_RING, s)

        for d in list(cws.values()) + list(ccws.values()):
            d.wait_send()
        for d in hbm_copies:
            d.wait()

    n_cw = len(_CW_IDX)
    n_ccw = len(_CCW_IDX)
    return pl.pallas_call(
        body,
        out_shape=jax.ShapeDtypeStruct((m, n), jnp.bfloat16),
        in_specs=[pl.BlockSpec(memory_space=pl.ANY)],
        out_specs=pl.BlockSpec(memory_space=pltpu.MemorySpace.HBM),
        scratch_shapes=[
            pltpu.VMEM((m, n), jnp.bfloat16),
            pltpu.VMEM((chunk, n), jnp.float32),
            pltpu.VMEM((chunk, n), jnp.bfloat16),
            pltpu.VMEM((chunk, n), jnp.bfloat16),
            pltpu.SemaphoreType.DMA,
            pltpu.SemaphoreType.DMA,
            pltpu.SemaphoreType.DMA((S,)),
            pltpu.SemaphoreType.DMA((S,)),
            pltpu.SemaphoreType.DMA((n_cw,)),
            pltpu.SemaphoreType.DMA((n_cw,)),
            pltpu.SemaphoreType.DMA((n_ccw,)),
            pltpu.SemaphoreType.DMA((n_ccw,)),
        ],
        compiler_params=pltpu.CompilerParams(collective_id=0),
    )(x)


# device time: 95623 ns/iter; 1.0096x vs baseline; 1.0096x over previous
import jax
import jax.numpy as jnp
from jax import lax
from jax.experimental import pallas as pl
from jax.experimental.pallas import tpu as pltpu

N_RING = 8
S = 8

_CW_HOPS = {s: (4 if s < S // 2 else 3) for s in range(S)}
_CCW_HOPS = {s: (3 if s < S // 2 else 4) for s in range(S)}

_CW_IDX = {(h, s): i for i, (h, s) in enumerate(
    (h, s) for h in range(4) for s in range(S) if h < _CW_HOPS[s])}
_CCW_IDX = {(h, s): i for i, (h, s) in enumerate(
    (h, s) for h in range(4) for s in range(S) if h < _CCW_HOPS[s])}


def kernel(x):
    m, n = x.shape
    chunk = m // N_RING
    sub = chunk // S

    def body(x_hbm, out_hbm, gather, xchunk, peer_chunk, send_buf,
             local_sem, hbm_sem, y_send, y_recv,
             cw_send, cw_recv, ccw_send, ccw_recv):
        my_x = lax.axis_index("x")
        my_y = lax.axis_index("y")
        my_z = lax.axis_index("z")

        p = jnp.where(my_x == 0, my_z, 7 - my_z)

        def ring_xz(q):
            q = q % N_RING
            return jnp.where(q <= 3, 0, 1), jnp.where(q <= 3, q, 7 - q)

        nxt_x, nxt_z = ring_xz(p + 1)
        prv_x, prv_z = ring_xz(p - 1)

        y_peer = (my_x, 1 - my_y, my_z)
        nxt = (nxt_x, my_y, nxt_z)
        prv = (prv_x, my_y, prv_z)

        loads = []
        for s in range(S):
            srows = pl.ds(s * sub, sub)
            cp = pltpu.make_async_copy(
                x_hbm.at[pl.ds(p * chunk + s * sub, sub)],
                xchunk.at[srows], local_sem.at[s])
            cp.start()
            loads.append(cp)

        bar = pltpu.get_barrier_semaphore()
        for dev in (y_peer, nxt, prv):
            pl.semaphore_signal(bar, inc=1, device_id=dev,
                                device_id_type=pl.DeviceIdType.MESH)
        pl.semaphore_wait(bar, 3)

        y_ex = []
        for s in range(S):
            srows = pl.ds(s * sub, sub)
            loads[s].wait()
            send_buf[srows, :] = xchunk[srows, :].astype(jnp.bfloat16)
            ex = pltpu.make_async_remote_copy(
                src_ref=send_buf.at[srows],
                dst_ref=peer_chunk.at[srows],
                send_sem=y_send.at[s], recv_sem=y_recv.at[s],
                device_id=y_peer, device_id_type=pl.DeviceIdType.MESH)
            ex.start()
            y_ex.append(ex)

        def mk(direction, h, s):
            if direction == "cw":
                c = (p - h) % N_RING
                dev, ssem, rsem, i = nxt, cw_send, cw_recv, _CW_IDX[(h, s)]
            else:
                c = (p + h) % N_RING
                dev, ssem, rsem, i = prv, ccw_send, ccw_recv, _CCW_IDX[(h, s)]
            off = c * chunk + s * sub
            return pltpu.make_async_remote_copy(
                src_ref=gather.at[pl.ds(off, sub)],
                dst_ref=gather.at[pl.ds(off, sub)],
                send_sem=ssem.at[i], recv_sem=rsem.at[i],
                device_id=dev, device_id_type=pl.DeviceIdType.MESH)

        hbm_copies = []

        def flush(c, s):
            off = c * chunk + s * sub
            d = pltpu.make_async_copy(
                gather.at[pl.ds(off, sub)], out_hbm.at[pl.ds(off, sub)],
                hbm_sem)
            d.start()
            hbm_copies.append(d)

        cws, ccws = {}, {}
        for s in range(S):
            y_ex[s].wait()
            rows = pl.ds(p * chunk + s * sub, sub)
            srows = pl.ds(s * sub, sub)
            gather[rows, :] = (
                xchunk[srows, :] + peer_chunk[srows, :].astype(jnp.float32)
            ).astype(jnp.bfloat16)
            cws[(0, s)] = mk("cw", 0, s)
            cws[(0, s)].start()
            ccws[(0, s)] = mk("ccw", 0, s)
            ccws[(0, s)].start()
            flush(p, s)

        for h in range(1, 4):
            for s in range(S):
                if h < _CW_HOPS[s]:
                    cws[(h - 1, s)].wait_recv()
                    cws[(h, s)] = mk("cw", h, s)
                    cws[(h, s)].start()
                    flush((p - h) % N_RING, s)
                if h < _CCW_HOPS[s]:
                    ccws[(h - 1, s)].wait_recv()
                    ccws[(h, s)] = mk("ccw", h, s)
                    ccws[(h, s)].start()
                    flush((p + h) % N_RING, s)

        for s in range(S):
            cws[(_CW_HOPS[s] - 1, s)].wait_recv()
            flush((p - _CW_HOPS[s]) % N_RING, s)
            ccws[(_CCW_HOPS[s] - 1, s)].wait_recv()
            flush((p + _CCW_HOPS[s]) % N_RING, s)

        for d in list(cws.values()) + list(ccws.values()):
            d.wait_send()
        for d in hbm_copies:
            d.wait()

    n_cw = len(_CW_IDX)
    n_ccw = len(_CCW_IDX)
    return pl.pallas_call(
        body,
        out_shape=jax.ShapeDtypeStruct((m, n), jnp.bfloat16),
        in_specs=[pl.BlockSpec(memory_space=pl.ANY)],
        out_specs=pl.BlockSpec(memory_space=pltpu.MemorySpace.HBM),
        scratch_shapes=[
            pltpu.VMEM((m, n), jnp.bfloat16),
            pltpu.VMEM((chunk, n), jnp.float32),
            pltpu.VMEM((chunk, n), jnp.bfloat16),
            pltpu.VMEM((chunk, n), jnp.bfloat16),
            pltpu.SemaphoreType.DMA((S,)),
            pltpu.SemaphoreType.DMA,
            pltpu.SemaphoreType.DMA((S,)),
            pltpu.SemaphoreType.DMA((S,)),
            pltpu.SemaphoreType.DMA((n_cw,)),
            pltpu.SemaphoreType.DMA((n_cw,)),
            pltpu.SemaphoreType.DMA((n_ccw,)),
            pltpu.SemaphoreType.DMA((n_ccw,)),
        ],
        compiler_params=pltpu.CompilerParams(collective_id=0),
    )(x)
